# baseline (speedup 1.0000x reference)
import jax
import jax.numpy as jnp
from jax.experimental import pallas as pl
from jax.experimental.pallas import tpu as pltpu

_MAX_TIME_MS = 600000.0
_ROW_BLK = 512
_NBUF = 4


def _fma_kernel(t_ref, w_ref, b_ref, o_hbm, buf, sems):
    i = pl.program_id(0)
    nblk = pl.num_programs(0)
    slot = jax.lax.rem(i, _NBUF)

    @pl.when(i >= _NBUF)
    def _wait_prev():
        prev = i - _NBUF
        pltpu.make_async_copy(
            buf.at[slot],
            o_hbm.at[pl.ds(prev * _ROW_BLK, _ROW_BLK), :],
            sems.at[slot],
        ).wait()

    w_scaled = w_ref[...] * (1.0 / _MAX_TIME_MS)
    t_blk = t_ref[pl.ds(i * _ROW_BLK, _ROW_BLK), :]
    buf[slot, :, :] = t_blk * w_scaled + b_ref[...]

    pltpu.make_async_copy(
        buf.at[slot],
        o_hbm.at[pl.ds(i * _ROW_BLK, _ROW_BLK), :],
        sems.at[slot],
    ).start()

    @pl.when(i == nblk - 1)
    def _drain():
        for j in range(_NBUF):
            k = nblk - _NBUF + j
            pltpu.make_async_copy(
                buf.at[j],
                o_hbm.at[pl.ds(k * _ROW_BLK, _ROW_BLK), :],
                sems.at[j],
            ).wait()


def kernel(timestamps, W, b):
    S, B, _ = timestamps.shape
    D = b.shape[0]
    n = S * B
    t2 = timestamps.reshape(n, 1)
    w_row = W.reshape(1, D)
    b_row = b.reshape(1, D)

    nblk = n // _ROW_BLK
    out = pl.pallas_call(
        _fma_kernel,
        grid=(nblk,),
        in_specs=[
            pl.BlockSpec((n, 1), lambda i: (0, 0)),
            pl.BlockSpec((1, D), lambda i: (0, 0)),
            pl.BlockSpec((1, D), lambda i: (0, 0)),
        ],
        out_specs=pl.BlockSpec(memory_space=pl.ANY),
        out_shape=jax.ShapeDtypeStruct((n, D), jnp.float32),
        scratch_shapes=[
            pltpu.VMEM((_NBUF, _ROW_BLK, D), jnp.float32),
            pltpu.SemaphoreType.DMA((_NBUF,)),
        ],
        compiler_params=pltpu.CompilerParams(
            dimension_semantics=("arbitrary",),
        ),
    )(t2, w_row, b_row)
    return out.reshape(S, B, D)
